# Initial kernel scaffold; baseline (speedup 1.0000x reference)
import jax
import jax.numpy as jnp
from jax.experimental import pallas as pl


def kernel(inputs, targets, centers):
    raise NotImplementedError("write your pallas kernel here")


# speedup vs baseline: 1.6769x; 1.6769x over previous
import functools

import jax
import jax.numpy as jnp
from jax import lax
from jax.experimental import pallas as pl
from jax.experimental.pallas import tpu as pltpu
from jax.experimental.pallas import tpu_sc as plsc

LANES = 16
CHUNK = 8


@functools.lru_cache(maxsize=None)
def _build_sc_kernel(B, D, n_workers):
    rows_per_w = B // n_workers
    n_chunks = rows_per_w // CHUNK
    n_outer = n_chunks // 2

    mesh = plsc.VectorSubcoreMesh(core_axis_name="c", subcore_axis_name="s")

    @functools.partial(
        pl.kernel,
        mesh=mesh,
        out_type=jax.ShapeDtypeStruct((n_workers, LANES), jnp.float32),
        scratch_types=[
            pltpu.VMEM((2, CHUNK, D), jnp.float32),
            pltpu.VMEM((2, CHUNK, D), jnp.float32),
            pltpu.VMEM((n_chunks, CHUNK), jnp.int32),
            pltpu.VMEM((LANES,), jnp.float32),
            pltpu.SemaphoreType.DMA,
            pltpu.SemaphoreType.DMA,
            pltpu.SemaphoreType.DMA,
            pltpu.SemaphoreType.DMA,
        ],
    )
    def sc_fn(x_hbm, t_hbm, cent_hbm, out_hbm,
              x_bufs, c_bufs, idx_all, acc_v, sx0, sx1, sc0, sc1):
        nc = 2
        wid = lax.axis_index("s") * nc + lax.axis_index("c")
        row0 = wid * rows_per_w

        pltpu.sync_copy(t_hbm.at[wid], idx_all)

        sx = (sx0, sx1)
        sc = (sc0, sc1)

        def start(chunk, buf):
            pltpu.async_copy(
                x_hbm.at[pl.ds(row0 + chunk * CHUNK, CHUNK)],
                x_bufs.at[buf], sx[buf])
            pltpu.async_copy(
                cent_hbm.at[idx_all.at[chunk]],
                c_bufs.at[buf], sc[buf])

        def wait(chunk, buf):
            pltpu.make_async_copy(
                x_hbm.at[pl.ds(row0, CHUNK)], x_bufs.at[buf], sx[buf]).wait()
            pltpu.make_async_copy(
                cent_hbm.at[idx_all.at[chunk]], c_bufs.at[buf], sc[buf]).wait()

        def accumulate(buf, accs):
            def body(j, accs):
                o = j * LANES
                new = []
                for r in range(CHUNK):
                    d = (x_bufs[buf, r, pl.ds(o, LANES)]
                         - c_bufs[buf, r, pl.ds(o, LANES)])
                    new.append(accs[r] + d * d)
                return tuple(new)
            return lax.fori_loop(0, D // LANES, body, accs)

        zero = jnp.zeros((LANES,), jnp.float32)
        accs0 = (zero,) * CHUNK

        start(0, 0)

        def outer(g, accs):
            ca = 2 * g
            cb = ca + 1
            start(cb, 1)
            wait(ca, 0)
            accs = accumulate(0, accs)

            @pl.when(g < n_outer - 1)
            def _():
                start(ca + 2, 0)

            wait(cb, 1)
            accs = accumulate(1, accs)
            return accs

        accs = lax.fori_loop(0, n_outer, outer, accs0)

        total = accs[0]
        for r in range(1, CHUNK):
            total = total + accs[r]
        acc_v[...] = total
        pltpu.sync_copy(acc_v, out_hbm.at[wid])

    return sc_fn


def kernel(inputs, targets, centers):
    B, D = inputs.shape
    info = plsc.get_sparse_core_info()
    n_workers = info.num_cores * info.num_subcores
    t = targets.astype(jnp.int32).reshape(n_workers, B // n_workers // CHUNK, CHUNK)
    partials = _build_sc_kernel(B, D, n_workers)(inputs, t, centers)
    return jnp.sum(partials) / B
